# baseline (speedup 1.0000x reference)
import jax
import jax.numpy as jnp
from jax import lax
from jax.experimental import pallas as pl
from jax.experimental.pallas import tpu as pltpu
from jax.experimental.pallas import tpu_sc as plsc

NC = 2
NS = 16
LANES = 16
C = 128
NBUF = 3


def _sc_mesh():
    return plsc.VectorSubcoreMesh(
        core_axis_name="c", subcore_axis_name="s", num_cores=NC, num_subcores=NS
    )


def _zero_rows(buf, nrows, width):
    @pl.loop(0, nrows)
    def _(j):
        for k in range(width // LANES):
            buf[j, pl.ds(k * LANES, LANES)] = jnp.zeros((LANES,), jnp.float32)


def _degrees(src_arr, dst_arr, n_pad):
    E = src_arr.shape[0]
    epw = E // NS
    n_full = epw // C
    n_grp = n_full // NBUF
    n_rem = n_full - n_grp * NBUF
    tail = epw - n_full * C
    rpw = n_pad // NS

    def body(src_ref, dst_ref, deg_out_ref, deg_in_ref, ifull, idx_t, ones_v,
             zbuf, acc, d0, d1, d2, s0, s1, s2):
        c = lax.axis_index("c")
        s = lax.axis_index("s")
        didx = [d0, d1, d2]
        ssem = [s0, s1, s2]

        @pl.loop(0, C // LANES)
        def _(k):
            ones_v[pl.ds(pl.multiple_of(k * LANES, LANES), LANES)] = jnp.ones(
                (LANES,), jnp.float32
            )

        @pl.loop(0, rpw // LANES)
        def _(k):
            zbuf[pl.ds(pl.multiple_of(k * LANES, LANES), LANES)] = jnp.zeros(
                (LANES,), jnp.float32
            )

        r0 = s * rpw
        e0 = s * epw

        def run(idx_hbm, out_hbm):
            pltpu.sync_copy(zbuf, acc.at[pl.ds(r0, rpw)])
            pltpu.sync_copy(idx_hbm.at[pl.ds(e0, epw)], ifull)
            plsc.subcore_barrier()

            def prep(b, off):
                for k in range(C // LANES):
                    didx[b][pl.ds(k * LANES, LANES)] = ifull[
                        pl.ds(off + k * LANES, LANES)
                    ]

            def chunk_group(g, nb):
                descs = []
                for b in range(nb):
                    off = pl.multiple_of(g * NBUF * C, C) + b * C
                    prep(b, off)
                    descs.append(
                        pltpu.async_copy(ones_v, acc.at[didx[b]], ssem[b],
                                         add=True)
                    )
                for d in descs:
                    d.wait()

            @pl.loop(0, n_grp)
            def _(g):
                chunk_group(g, NBUF)

            if n_rem:
                chunk_group(n_grp, n_rem)

            if tail:
                pltpu.sync_copy(idx_hbm.at[pl.ds(e0 + n_full * C, tail)], idx_t)
                pltpu.sync_copy(ones_v.at[pl.ds(0, tail)], acc.at[idx_t],
                                add=True)

            plsc.subcore_barrier()
            pltpu.sync_copy(acc.at[pl.ds(r0, rpw)], out_hbm.at[pl.ds(r0, rpw)])

        @pl.when(c == 0)
        def _():
            run(src_ref, deg_out_ref)

        @pl.when(c == 1)
        def _():
            run(dst_ref, deg_in_ref)

    f = pl.kernel(
        body,
        out_type=[
            jax.ShapeDtypeStruct((n_pad,), jnp.float32),
            jax.ShapeDtypeStruct((n_pad,), jnp.float32),
        ],
        mesh=_sc_mesh(),
        scratch_types=[
            pltpu.VMEM((epw,), jnp.int32),
            pltpu.VMEM((16,), jnp.int32),
            pltpu.VMEM((C,), jnp.float32),
            pltpu.VMEM((rpw,), jnp.float32),
            pltpu.VMEM_SHARED((n_pad,), jnp.float32),
            pltpu.VMEM((C,), jnp.int32),
            pltpu.VMEM((C,), jnp.int32),
            pltpu.VMEM((C,), jnp.int32),
            pltpu.SemaphoreType.DMA,
            pltpu.SemaphoreType.DMA,
            pltpu.SemaphoreType.DMA,
        ],
    )
    return f(src_arr, dst_arr)


def _spmm_sc(hw2d, src_arr, dst_arr, n_pad, split_features, cs=96, nbuf=NBUF):
    E = src_arr.shape[0]
    dh = hw2d.shape[1]
    half_m = hw2d.shape[0] // 2
    nworkers = NS if split_features else NC * NS
    epw = E // nworkers
    n_full = epw // cs
    n_grp = n_full // nbuf
    n_rem = n_full - n_grp * nbuf
    tail = epw - n_full * cs
    rpw = n_pad // NS

    def body(hw_ref, src_ref, dst_ref, out_a_ref, out_b_ref, sfull,
             sidx_t, didx_t, gidx_t, acc, *scr):
        c = lax.axis_index("c")
        s = lax.axis_index("s")
        rows = list(scr[0:nbuf])
        gidx = list(scr[nbuf:2 * nbuf])
        didx = list(scr[2 * nbuf:3 * nbuf])
        gsem = list(scr[3 * nbuf:4 * nbuf])
        dsem = list(scr[4 * nbuf:5 * nbuf])
        ssem = list(scr[5 * nbuf:6 * nbuf])

        _zero_rows(rows[0], cs, dh)
        r0 = s * rpw
        nz = rpw // cs
        for b in range(nz):
            pltpu.sync_copy(rows[0], acc.at[pl.ds(r0 + b * cs, cs)])
        if rpw - nz * cs:
            pltpu.sync_copy(rows[0].at[pl.ds(0, rpw - nz * cs)],
                            acc.at[pl.ds(r0 + nz * cs, rpw - nz * cs)])

        e0 = (s if split_features else c * NS + s) * epw
        pltpu.sync_copy(src_ref.at[pl.ds(e0, epw)], sfull)
        plsc.subcore_barrier()

        def chunk_group(g, nb):
            gdescs = []
            ddescs = []
            for b in range(nb):
                off = pl.multiple_of(g * nbuf * cs, 8) + b * cs
                ddescs.append(
                    pltpu.async_copy(dst_ref.at[pl.ds(e0 + off, cs)], didx[b],
                                     dsem[b])
                )
                if split_features:
                    for k in range(cs // LANES):
                        gidx[b][pl.ds(k * LANES, LANES)] = (
                            sfull[pl.ds(off + k * LANES, LANES)] + c * half_m
                        )
                    isrc = gidx[b]
                else:
                    isrc = sfull.at[pl.ds(off, cs)]
                gdescs.append(
                    pltpu.async_copy(hw_ref.at[isrc], rows[b], gsem[b])
                )
            sdescs = []
            for b in range(nb):
                gdescs[b].wait()
                ddescs[b].wait()
                sdescs.append(
                    pltpu.async_copy(rows[b], acc.at[didx[b]], ssem[b],
                                     add=True)
                )
            for d in sdescs:
                d.wait()

        @pl.loop(0, n_grp)
        def _(g):
            chunk_group(g, nbuf)

        if n_rem:
            chunk_group(n_grp, n_rem)

        if tail:
            toff = e0 + n_full * cs
            pltpu.sync_copy(src_ref.at[pl.ds(toff, tail)], sidx_t)
            pltpu.sync_copy(dst_ref.at[pl.ds(toff, tail)], didx_t)
            if split_features:
                assert tail % LANES == 0
                for k in range(tail // LANES):
                    li = pl.ds(k * LANES, LANES)
                    gidx_t[li] = sidx_t[li] + c * half_m
                gsrc = gidx_t
            else:
                gsrc = sidx_t
            dst = rows[0].at[pl.ds(0, tail)]
            pltpu.async_copy(hw_ref.at[gsrc], dst, gsem[0]).wait()
            pltpu.sync_copy(dst, acc.at[didx_t], add=True)

        plsc.subcore_barrier()

        @pl.when(c == 0)
        def _():
            pltpu.sync_copy(acc.at[pl.ds(r0, rpw)], out_a_ref.at[pl.ds(r0, rpw)])

        @pl.when(c == 1)
        def _():
            pltpu.sync_copy(acc.at[pl.ds(r0, rpw)], out_b_ref.at[pl.ds(r0, rpw)])

    tb = tail if tail else LANES
    scratch = (
        [
            pltpu.VMEM((epw,), jnp.int32),
            pltpu.VMEM((tb,), jnp.int32),
            pltpu.VMEM((tb,), jnp.int32),
            pltpu.VMEM((tb,), jnp.int32),
            pltpu.VMEM_SHARED((n_pad, dh), jnp.float32),
        ]
        + [pltpu.VMEM((cs, dh), jnp.float32)] * nbuf
        + [pltpu.VMEM((cs,), jnp.int32)] * (2 * nbuf)
        + [pltpu.SemaphoreType.DMA] * (3 * nbuf)
    )
    f = pl.kernel(
        body,
        out_type=[
            jax.ShapeDtypeStruct((n_pad, dh), jnp.float32),
            jax.ShapeDtypeStruct((n_pad, dh), jnp.float32),
        ],
        mesh=_sc_mesh(),
        scratch_types=scratch,
    )
    return f(hw2d, src_arr, dst_arr)


def _tc_norm_matmul(x, deg_out, W):
    n, d_in = x.shape
    d_h = W.shape[1]
    dhh = d_h // 2
    br = 2000
    assert n % br == 0

    def body(x_ref, d_ref, w_ref, o_ref):
        nrm = lax.rsqrt(jnp.maximum(d_ref[...], 1.0))
        xn = x_ref[...] * nrm
        o_ref[0, ...] = jnp.dot(
            xn, w_ref[:, :dhh], preferred_element_type=jnp.float32
        )
        o_ref[1, ...] = jnp.dot(
            xn, w_ref[:, dhh:], preferred_element_type=jnp.float32
        )

    return pl.pallas_call(
        body,
        grid=(n // br,),
        in_specs=[
            pl.BlockSpec((br, d_in), lambda i: (i, 0)),
            pl.BlockSpec((br, 1), lambda i: (i, 0)),
            pl.BlockSpec((d_in, d_h), lambda i: (0, 0)),
        ],
        out_specs=pl.BlockSpec((2, br, dhh), lambda i: (0, i, 0)),
        out_shape=jax.ShapeDtypeStruct((2, n, dhh), jnp.float32),
    )(x, deg_out, W)


def _tc_mid(agg_a, agg_b, deg_in, deg_out, b1, W2):
    n, dhh = agg_a.shape
    d_h = 2 * dhh
    d_out = W2.shape[1]
    br = 1280
    assert n % br == 0

    d_pad = 128

    def body(a_ref, bh_ref, di_ref, do_ref, b_ref, w_ref, o_ref):
        ni = lax.rsqrt(jnp.maximum(di_ref[...], 1.0))
        no = lax.rsqrt(jnp.maximum(do_ref[...], 1.0))
        h = jnp.concatenate([a_ref[...], bh_ref[...]], axis=1)
        h = jnp.maximum(h * ni + b_ref[...], 0.0) * no
        hw = jnp.dot(h, w_ref[...], preferred_element_type=jnp.float32)
        o_ref[...] = jnp.concatenate(
            [hw, jnp.zeros((br, d_pad - d_out), jnp.float32)], axis=1
        )

    return pl.pallas_call(
        body,
        grid=(n // br,),
        in_specs=[
            pl.BlockSpec((br, dhh), lambda i: (i, 0)),
            pl.BlockSpec((br, dhh), lambda i: (i, 0)),
            pl.BlockSpec((br, 1), lambda i: (i, 0)),
            pl.BlockSpec((br, 1), lambda i: (i, 0)),
            pl.BlockSpec((1, d_h), lambda i: (0, 0)),
            pl.BlockSpec((d_h, d_out), lambda i: (0, 0)),
        ],
        out_specs=pl.BlockSpec((br, d_pad), lambda i: (i, 0)),
        out_shape=jax.ShapeDtypeStruct((n, d_pad), jnp.float32),
    )(agg_a, agg_b, deg_in, deg_out, b1.reshape(1, d_h), W2)


def _tc_final(part_a, part_b, deg_in2, b2, n):
    npad, dpad = part_a.shape
    d_out = b2.shape[0]
    br = 2000
    assert n % br == 0

    def body(a_ref, bh_ref, di_ref, b_ref, o_ref):
        ni = lax.rsqrt(jnp.maximum(di_ref[...], 1.0))
        a2 = a_ref[...] + bh_ref[...]
        o_ref[...] = a2[:, :d_out] * ni + b_ref[...]

    return pl.pallas_call(
        body,
        grid=(n // br,),
        in_specs=[
            pl.BlockSpec((br, dpad), lambda i: (i, 0)),
            pl.BlockSpec((br, dpad), lambda i: (i, 0)),
            pl.BlockSpec((br, 1), lambda i: (i, 0)),
            pl.BlockSpec((1, d_out), lambda i: (0, 0)),
        ],
        out_specs=pl.BlockSpec((br, d_out), lambda i: (i, 0)),
        out_shape=jax.ShapeDtypeStruct((n, d_out), jnp.float32),
    )(part_a, part_b, deg_in2, b2.reshape(1, d_out))


def kernel(in_feat, edge_index, W1, b1, W2, b2):
    n, d_in = in_feat.shape
    e = edge_index.shape[1]
    d_h = W1.shape[1]
    d_out = W2.shape[1]
    assert e % (NC * NS) == 0 and d_h % 2 == 0 and d_out % 2 == 0

    n_pad = -(-n // (NS * C)) * (NS * C)

    src_arr = edge_index[0]
    dst_arr = edge_index[1]
    deg_out, deg_in = _degrees(src_arr, dst_arr, n_pad)
    deg_out = deg_out.reshape(n_pad, 1)
    deg_in = deg_in.reshape(n_pad, 1)

    hw1 = _tc_norm_matmul(in_feat, deg_out, W1)
    a1, a1b = _spmm_sc(hw1.reshape(2 * n, d_h // 2), src_arr, dst_arr, n_pad,
                       split_features=True, cs=128, nbuf=2)

    hw2 = _tc_mid(a1, a1b, deg_in, deg_out, b1, W2)
    p2a, p2b = _spmm_sc(hw2, src_arr, dst_arr, n_pad, split_features=False,
                        cs=128, nbuf=2)
    return _tc_final(p2a, p2b, deg_in, b2, n)


# speedup vs baseline: 1.0278x; 1.0278x over previous
import jax
import jax.numpy as jnp
from jax import lax
from jax.experimental import pallas as pl
from jax.experimental.pallas import tpu as pltpu
from jax.experimental.pallas import tpu_sc as plsc

NC = 2
NS = 16
LANES = 16
C = 128
NBUF = 3


def _sc_mesh():
    return plsc.VectorSubcoreMesh(
        core_axis_name="c", subcore_axis_name="s", num_cores=NC, num_subcores=NS
    )


def _zero_rows(buf, nrows, width):
    @pl.loop(0, nrows)
    def _(j):
        for k in range(width // LANES):
            buf[j, pl.ds(k * LANES, LANES)] = jnp.zeros((LANES,), jnp.float32)


def _degrees(src_arr, dst_arr, n_pad):
    E = src_arr.shape[0]
    epw = E // NS
    n_full = epw // C
    n_grp = n_full // NBUF
    n_rem = n_full - n_grp * NBUF
    tail = epw - n_full * C
    rpw = n_pad // NS

    def body(src_ref, dst_ref, deg_out_ref, deg_in_ref, ifull, idx_t, ones_v,
             zbuf, acc, d0, d1, d2, s0, s1, s2):
        c = lax.axis_index("c")
        s = lax.axis_index("s")
        didx = [d0, d1, d2]
        ssem = [s0, s1, s2]

        @pl.loop(0, C // LANES)
        def _(k):
            ones_v[pl.ds(pl.multiple_of(k * LANES, LANES), LANES)] = jnp.ones(
                (LANES,), jnp.float32
            )

        @pl.loop(0, rpw // LANES)
        def _(k):
            zbuf[pl.ds(pl.multiple_of(k * LANES, LANES), LANES)] = jnp.zeros(
                (LANES,), jnp.float32
            )

        r0 = s * rpw
        e0 = s * epw

        def run(idx_hbm, out_hbm):
            pltpu.sync_copy(zbuf, acc.at[pl.ds(r0, rpw)])
            pltpu.sync_copy(idx_hbm.at[pl.ds(e0, epw)], ifull)
            plsc.subcore_barrier()

            def prep(b, off):
                for k in range(C // LANES):
                    didx[b][pl.ds(k * LANES, LANES)] = ifull[
                        pl.ds(off + k * LANES, LANES)
                    ]

            def chunk_group(g, nb):
                descs = []
                for b in range(nb):
                    off = pl.multiple_of(g * NBUF * C, C) + b * C
                    prep(b, off)
                    descs.append(
                        pltpu.async_copy(ones_v, acc.at[didx[b]], ssem[b],
                                         add=True)
                    )
                for d in descs:
                    d.wait()

            @pl.loop(0, n_grp)
            def _(g):
                chunk_group(g, NBUF)

            if n_rem:
                chunk_group(n_grp, n_rem)

            if tail:
                pltpu.sync_copy(idx_hbm.at[pl.ds(e0 + n_full * C, tail)], idx_t)
                pltpu.sync_copy(ones_v.at[pl.ds(0, tail)], acc.at[idx_t],
                                add=True)

            plsc.subcore_barrier()
            pltpu.sync_copy(acc.at[pl.ds(r0, rpw)], out_hbm.at[pl.ds(r0, rpw)])

        @pl.when(c == 0)
        def _():
            run(src_ref, deg_out_ref)

        @pl.when(c == 1)
        def _():
            run(dst_ref, deg_in_ref)

    f = pl.kernel(
        body,
        out_type=[
            jax.ShapeDtypeStruct((n_pad,), jnp.float32),
            jax.ShapeDtypeStruct((n_pad,), jnp.float32),
        ],
        mesh=_sc_mesh(),
        scratch_types=[
            pltpu.VMEM((epw,), jnp.int32),
            pltpu.VMEM((16,), jnp.int32),
            pltpu.VMEM((C,), jnp.float32),
            pltpu.VMEM((rpw,), jnp.float32),
            pltpu.VMEM_SHARED((n_pad,), jnp.float32),
            pltpu.VMEM((C,), jnp.int32),
            pltpu.VMEM((C,), jnp.int32),
            pltpu.VMEM((C,), jnp.int32),
            pltpu.SemaphoreType.DMA,
            pltpu.SemaphoreType.DMA,
            pltpu.SemaphoreType.DMA,
        ],
    )
    return f(src_arr, dst_arr)


def _spmm_sc(hw2d, src_arr, dst_arr, n_pad, split_features, cs=96, nbuf=NBUF):
    E = src_arr.shape[0]
    dh = hw2d.shape[1]
    half_m = hw2d.shape[0] // 2
    nworkers = NS if split_features else NC * NS
    epw = E // nworkers
    n_full = epw // cs
    n_grp = n_full // nbuf
    n_rem = n_full - n_grp * nbuf
    tail = epw - n_full * cs
    rpw = n_pad // NS

    def body(hw_ref, src_ref, dst_ref, out_a_ref, out_b_ref, sfull,
             sidx_t, didx_t, gidx_t, acc, *scr):
        c = lax.axis_index("c")
        s = lax.axis_index("s")
        rows = list(scr[0:nbuf])
        gidx = list(scr[nbuf:2 * nbuf])
        didx = list(scr[2 * nbuf:3 * nbuf])
        gsem = list(scr[3 * nbuf:4 * nbuf])
        dsem = list(scr[4 * nbuf:5 * nbuf])
        ssem = list(scr[5 * nbuf:6 * nbuf])

        _zero_rows(rows[0], cs, dh)
        r0 = s * rpw
        nz = rpw // cs
        for b in range(nz):
            pltpu.sync_copy(rows[0], acc.at[pl.ds(r0 + b * cs, cs)])
        if rpw - nz * cs:
            pltpu.sync_copy(rows[0].at[pl.ds(0, rpw - nz * cs)],
                            acc.at[pl.ds(r0 + nz * cs, rpw - nz * cs)])

        e0 = (s if split_features else c * NS + s) * epw
        pltpu.sync_copy(src_ref.at[pl.ds(e0, epw)], sfull)
        plsc.subcore_barrier()

        def chunk_group(g, nb):
            gdescs = []
            ddescs = []
            for b in range(nb):
                off = pl.multiple_of(g * nbuf * cs, 8) + b * cs
                ddescs.append(
                    pltpu.async_copy(dst_ref.at[pl.ds(e0 + off, cs)], didx[b],
                                     dsem[b])
                )
                if split_features:
                    for k in range(cs // LANES):
                        gidx[b][pl.ds(k * LANES, LANES)] = (
                            sfull[pl.ds(off + k * LANES, LANES)] + c * half_m
                        )
                    isrc = gidx[b]
                else:
                    isrc = sfull.at[pl.ds(off, cs)]
                gdescs.append(
                    pltpu.async_copy(hw_ref.at[isrc], rows[b], gsem[b])
                )
            sdescs = []
            for b in range(nb):
                gdescs[b].wait()
                ddescs[b].wait()
                sdescs.append(
                    pltpu.async_copy(rows[b], acc.at[didx[b]], ssem[b],
                                     add=True)
                )
            for d in sdescs:
                d.wait()

        @pl.loop(0, n_grp)
        def _(g):
            chunk_group(g, nbuf)

        if n_rem:
            chunk_group(n_grp, n_rem)

        if tail:
            toff = e0 + n_full * cs
            pltpu.sync_copy(src_ref.at[pl.ds(toff, tail)], sidx_t)
            pltpu.sync_copy(dst_ref.at[pl.ds(toff, tail)], didx_t)
            if split_features:
                assert tail % LANES == 0
                for k in range(tail // LANES):
                    li = pl.ds(k * LANES, LANES)
                    gidx_t[li] = sidx_t[li] + c * half_m
                gsrc = gidx_t
            else:
                gsrc = sidx_t
            dst = rows[0].at[pl.ds(0, tail)]
            pltpu.async_copy(hw_ref.at[gsrc], dst, gsem[0]).wait()
            pltpu.sync_copy(dst, acc.at[didx_t], add=True)

        plsc.subcore_barrier()

        @pl.when(c == 0)
        def _():
            pltpu.sync_copy(acc.at[pl.ds(r0, rpw)], out_a_ref.at[pl.ds(r0, rpw)])

        @pl.when(c == 1)
        def _():
            pltpu.sync_copy(acc.at[pl.ds(r0, rpw)], out_b_ref.at[pl.ds(r0, rpw)])

    tb = tail if tail else LANES
    scratch = (
        [
            pltpu.VMEM((epw,), jnp.int32),
            pltpu.VMEM((tb,), jnp.int32),
            pltpu.VMEM((tb,), jnp.int32),
            pltpu.VMEM((tb,), jnp.int32),
            pltpu.VMEM_SHARED((n_pad, dh), jnp.float32),
        ]
        + [pltpu.VMEM((cs, dh), jnp.float32)] * nbuf
        + [pltpu.VMEM((cs,), jnp.int32)] * (2 * nbuf)
        + [pltpu.SemaphoreType.DMA] * (3 * nbuf)
    )
    f = pl.kernel(
        body,
        out_type=[
            jax.ShapeDtypeStruct((n_pad, dh), jnp.float32),
            jax.ShapeDtypeStruct((n_pad, dh), jnp.float32),
        ],
        mesh=_sc_mesh(),
        scratch_types=scratch,
    )
    return f(hw2d, src_arr, dst_arr)


def _tc_norm_matmul(x, deg_out, W):
    n, d_in = x.shape
    d_h = W.shape[1]
    dhh = d_h // 2
    br = 2000
    assert n % br == 0

    def body(x_ref, d_ref, w_ref, o_ref):
        nrm = lax.rsqrt(jnp.maximum(d_ref[...], 1.0))
        xn = x_ref[...] * nrm
        o_ref[0, ...] = jnp.dot(
            xn, w_ref[:, :dhh], preferred_element_type=jnp.float32
        )
        o_ref[1, ...] = jnp.dot(
            xn, w_ref[:, dhh:], preferred_element_type=jnp.float32
        )

    return pl.pallas_call(
        body,
        grid=(n // br,),
        in_specs=[
            pl.BlockSpec((br, d_in), lambda i: (i, 0)),
            pl.BlockSpec((br, 1), lambda i: (i, 0)),
            pl.BlockSpec((d_in, d_h), lambda i: (0, 0)),
        ],
        out_specs=pl.BlockSpec((2, br, dhh), lambda i: (0, i, 0)),
        out_shape=jax.ShapeDtypeStruct((2, n, dhh), jnp.float32),
    )(x, deg_out, W)


def _tc_mid(agg_a, agg_b, deg_in, deg_out, b1, W2):
    n, dhh = agg_a.shape
    d_h = 2 * dhh
    d_out = W2.shape[1]
    br = 1280
    assert n % br == 0

    d_pad = 128

    def body(a_ref, bh_ref, di_ref, do_ref, b_ref, w_ref, o_ref):
        ni = lax.rsqrt(jnp.maximum(di_ref[...], 1.0))
        no = lax.rsqrt(jnp.maximum(do_ref[...], 1.0))
        h = jnp.concatenate([a_ref[...], bh_ref[...]], axis=1)
        h = jnp.maximum(h * ni + b_ref[...], 0.0) * no
        hw = jnp.dot(h, w_ref[...], preferred_element_type=jnp.float32)
        o_ref[...] = jnp.concatenate(
            [hw, jnp.zeros((br, d_pad - d_out), jnp.float32)], axis=1
        )

    return pl.pallas_call(
        body,
        grid=(n // br,),
        in_specs=[
            pl.BlockSpec((br, dhh), lambda i: (i, 0)),
            pl.BlockSpec((br, dhh), lambda i: (i, 0)),
            pl.BlockSpec((br, 1), lambda i: (i, 0)),
            pl.BlockSpec((br, 1), lambda i: (i, 0)),
            pl.BlockSpec((1, d_h), lambda i: (0, 0)),
            pl.BlockSpec((d_h, d_out), lambda i: (0, 0)),
        ],
        out_specs=pl.BlockSpec((br, d_pad), lambda i: (i, 0)),
        out_shape=jax.ShapeDtypeStruct((n, d_pad), jnp.float32),
    )(agg_a, agg_b, deg_in, deg_out, b1.reshape(1, d_h), W2)


def _tc_final(part_a, part_b, deg_in2, b2, n):
    npad, dpad = part_a.shape
    d_out = b2.shape[0]
    br = 2000
    assert n % br == 0

    def body(a_ref, bh_ref, di_ref, b_ref, o_ref):
        ni = lax.rsqrt(jnp.maximum(di_ref[...], 1.0))
        a2 = a_ref[...] + bh_ref[...]
        o_ref[...] = a2[:, :d_out] * ni + b_ref[...]

    return pl.pallas_call(
        body,
        grid=(n // br,),
        in_specs=[
            pl.BlockSpec((br, dpad), lambda i: (i, 0)),
            pl.BlockSpec((br, dpad), lambda i: (i, 0)),
            pl.BlockSpec((br, 1), lambda i: (i, 0)),
            pl.BlockSpec((1, d_out), lambda i: (0, 0)),
        ],
        out_specs=pl.BlockSpec((br, d_out), lambda i: (i, 0)),
        out_shape=jax.ShapeDtypeStruct((n, d_out), jnp.float32),
    )(part_a, part_b, deg_in2, b2.reshape(1, d_out))


def kernel(in_feat, edge_index, W1, b1, W2, b2):
    n, d_in = in_feat.shape
    e = edge_index.shape[1]
    d_h = W1.shape[1]
    d_out = W2.shape[1]
    assert e % (NC * NS) == 0 and d_h % 2 == 0 and d_out % 2 == 0

    n_pad = -(-n // (NS * C)) * (NS * C)

    src_arr = edge_index[0]
    dst_arr = edge_index[1]
    deg_out, deg_in = _degrees(src_arr, dst_arr, n_pad)
    deg_out = deg_out.reshape(n_pad, 1)
    deg_in = deg_in.reshape(n_pad, 1)

    hw1 = _tc_norm_matmul(in_feat, deg_out, W1)
    a1, a1b = _spmm_sc(hw1.reshape(2 * n, d_h // 2), src_arr, dst_arr, n_pad,
                       split_features=True, cs=64, nbuf=4)

    hw2 = _tc_mid(a1, a1b, deg_in, deg_out, b1, W2)
    p2a, p2b = _spmm_sc(hw2, src_arr, dst_arr, n_pad, split_features=False,
                        cs=64, nbuf=4)
    return _tc_final(p2a, p2b, deg_in, b2, n)


# speedup vs baseline: 1.0335x; 1.0055x over previous
import jax
import jax.numpy as jnp
from jax import lax
from jax.experimental import pallas as pl
from jax.experimental.pallas import tpu as pltpu
from jax.experimental.pallas import tpu_sc as plsc

NC = 2
NS = 16
LANES = 16
C = 128
NBUF = 3


def _sc_mesh():
    return plsc.VectorSubcoreMesh(
        core_axis_name="c", subcore_axis_name="s", num_cores=NC, num_subcores=NS
    )


def _zero_rows(buf, nrows, width):
    @pl.loop(0, nrows)
    def _(j):
        for k in range(width // LANES):
            buf[j, pl.ds(k * LANES, LANES)] = jnp.zeros((LANES,), jnp.float32)


def _degrees(src_arr, dst_arr, n_pad):
    E = src_arr.shape[0]
    epw = E // NS
    n_full = epw // C
    n_grp = n_full // NBUF
    n_rem = n_full - n_grp * NBUF
    tail = epw - n_full * C
    rpw = n_pad // NS

    def body(src_ref, dst_ref, deg_out_ref, deg_in_ref, ifull, idx_t, ones_v,
             zbuf, acc, d0, d1, d2, s0, s1, s2):
        c = lax.axis_index("c")
        s = lax.axis_index("s")
        didx = [d0, d1, d2]
        ssem = [s0, s1, s2]

        @pl.loop(0, C // LANES)
        def _(k):
            ones_v[pl.ds(pl.multiple_of(k * LANES, LANES), LANES)] = jnp.ones(
                (LANES,), jnp.float32
            )

        @pl.loop(0, rpw // LANES)
        def _(k):
            zbuf[pl.ds(pl.multiple_of(k * LANES, LANES), LANES)] = jnp.zeros(
                (LANES,), jnp.float32
            )

        r0 = s * rpw
        e0 = s * epw

        def run(idx_hbm, out_hbm):
            pltpu.sync_copy(zbuf, acc.at[pl.ds(r0, rpw)])
            pltpu.sync_copy(idx_hbm.at[pl.ds(e0, epw)], ifull)
            plsc.subcore_barrier()

            def prep(b, off):
                for k in range(C // LANES):
                    didx[b][pl.ds(k * LANES, LANES)] = ifull[
                        pl.ds(off + k * LANES, LANES)
                    ]

            def chunk_group(g, nb):
                descs = []
                for b in range(nb):
                    off = pl.multiple_of(g * NBUF * C, C) + b * C
                    prep(b, off)
                    descs.append(
                        pltpu.async_copy(ones_v, acc.at[didx[b]], ssem[b],
                                         add=True)
                    )
                for d in descs:
                    d.wait()

            @pl.loop(0, n_grp)
            def _(g):
                chunk_group(g, NBUF)

            if n_rem:
                chunk_group(n_grp, n_rem)

            if tail:
                pltpu.sync_copy(idx_hbm.at[pl.ds(e0 + n_full * C, tail)], idx_t)
                pltpu.sync_copy(ones_v.at[pl.ds(0, tail)], acc.at[idx_t],
                                add=True)

            plsc.subcore_barrier()
            pltpu.sync_copy(acc.at[pl.ds(r0, rpw)], out_hbm.at[pl.ds(r0, rpw)])

        @pl.when(c == 0)
        def _():
            run(src_ref, deg_out_ref)

        @pl.when(c == 1)
        def _():
            run(dst_ref, deg_in_ref)

    f = pl.kernel(
        body,
        out_type=[
            jax.ShapeDtypeStruct((n_pad,), jnp.float32),
            jax.ShapeDtypeStruct((n_pad,), jnp.float32),
        ],
        mesh=_sc_mesh(),
        scratch_types=[
            pltpu.VMEM((epw,), jnp.int32),
            pltpu.VMEM((16,), jnp.int32),
            pltpu.VMEM((C,), jnp.float32),
            pltpu.VMEM((rpw,), jnp.float32),
            pltpu.VMEM_SHARED((n_pad,), jnp.float32),
            pltpu.VMEM((C,), jnp.int32),
            pltpu.VMEM((C,), jnp.int32),
            pltpu.VMEM((C,), jnp.int32),
            pltpu.SemaphoreType.DMA,
            pltpu.SemaphoreType.DMA,
            pltpu.SemaphoreType.DMA,
        ],
    )
    return f(src_arr, dst_arr)


def _spmm_sc(hw2d, src_arr, dst_arr, n_pad, split_features, cs=96, nbuf=NBUF):
    E = src_arr.shape[0]
    dh = hw2d.shape[1]
    half_m = hw2d.shape[0] // 2
    nworkers = NS if split_features else NC * NS
    epw = E // nworkers
    n_full = epw // cs
    n_grp = n_full // nbuf
    n_rem = n_full - n_grp * nbuf
    tail = epw - n_full * cs
    rpw = n_pad // NS

    def body(hw_ref, src_ref, dst_ref, out_a_ref, out_b_ref, sfull,
             sidx_t, didx_t, gidx_t, acc, *scr):
        c = lax.axis_index("c")
        s = lax.axis_index("s")
        rows = list(scr[0:nbuf])
        gidx = list(scr[nbuf:2 * nbuf])
        didx = list(scr[2 * nbuf:3 * nbuf])
        gsem = list(scr[3 * nbuf:4 * nbuf])
        dsem = list(scr[4 * nbuf:5 * nbuf])
        ssem = list(scr[5 * nbuf:6 * nbuf])

        _zero_rows(rows[0], cs, dh)
        r0 = s * rpw
        nz = rpw // cs
        for b in range(nz):
            pltpu.sync_copy(rows[0], acc.at[pl.ds(r0 + b * cs, cs)])
        if rpw - nz * cs:
            pltpu.sync_copy(rows[0].at[pl.ds(0, rpw - nz * cs)],
                            acc.at[pl.ds(r0 + nz * cs, rpw - nz * cs)])

        e0 = (s if split_features else c * NS + s) * epw
        pltpu.sync_copy(src_ref.at[pl.ds(e0, epw)], sfull)
        plsc.subcore_barrier()

        def chunk_group(g, nb):
            gdescs = []
            ddescs = []
            for b in range(nb):
                off = pl.multiple_of(g * nbuf * cs, 8) + b * cs
                ddescs.append(
                    pltpu.async_copy(dst_ref.at[pl.ds(e0 + off, cs)], didx[b],
                                     dsem[b])
                )
                if split_features:
                    for k in range(cs // LANES):
                        gidx[b][pl.ds(k * LANES, LANES)] = (
                            sfull[pl.ds(off + k * LANES, LANES)] + c * half_m
                        )
                    isrc = gidx[b]
                else:
                    isrc = sfull.at[pl.ds(off, cs)]
                gdescs.append(
                    pltpu.async_copy(hw_ref.at[isrc], rows[b], gsem[b])
                )
            sdescs = []
            for b in range(nb):
                gdescs[b].wait()
                ddescs[b].wait()
                sdescs.append(
                    pltpu.async_copy(rows[b], acc.at[didx[b]], ssem[b],
                                     add=True)
                )
            for d in sdescs:
                d.wait()

        @pl.loop(0, n_grp)
        def _(g):
            chunk_group(g, nbuf)

        if n_rem:
            chunk_group(n_grp, n_rem)

        if tail:
            toff = e0 + n_full * cs
            pltpu.sync_copy(src_ref.at[pl.ds(toff, tail)], sidx_t)
            pltpu.sync_copy(dst_ref.at[pl.ds(toff, tail)], didx_t)
            if split_features:
                assert tail % LANES == 0
                for k in range(tail // LANES):
                    li = pl.ds(k * LANES, LANES)
                    gidx_t[li] = sidx_t[li] + c * half_m
                gsrc = gidx_t
            else:
                gsrc = sidx_t
            dst = rows[0].at[pl.ds(0, tail)]
            pltpu.async_copy(hw_ref.at[gsrc], dst, gsem[0]).wait()
            pltpu.sync_copy(dst, acc.at[didx_t], add=True)

        plsc.subcore_barrier()

        @pl.when(c == 0)
        def _():
            pltpu.sync_copy(acc.at[pl.ds(r0, rpw)], out_a_ref.at[pl.ds(r0, rpw)])

        @pl.when(c == 1)
        def _():
            pltpu.sync_copy(acc.at[pl.ds(r0, rpw)], out_b_ref.at[pl.ds(r0, rpw)])

    tb = tail if tail else LANES
    scratch = (
        [
            pltpu.VMEM((epw,), jnp.int32),
            pltpu.VMEM((tb,), jnp.int32),
            pltpu.VMEM((tb,), jnp.int32),
            pltpu.VMEM((tb,), jnp.int32),
            pltpu.VMEM_SHARED((n_pad, dh), jnp.float32),
        ]
        + [pltpu.VMEM((cs, dh), jnp.float32)] * nbuf
        + [pltpu.VMEM((cs,), jnp.int32)] * (2 * nbuf)
        + [pltpu.SemaphoreType.DMA] * (3 * nbuf)
    )
    f = pl.kernel(
        body,
        out_type=[
            jax.ShapeDtypeStruct((n_pad, dh), jnp.float32),
            jax.ShapeDtypeStruct((n_pad, dh), jnp.float32),
        ],
        mesh=_sc_mesh(),
        scratch_types=scratch,
    )
    return f(hw2d, src_arr, dst_arr)


def _tc_norm_matmul(x, deg_out, W):
    n, d_in = x.shape
    d_h = W.shape[1]
    dhh = d_h // 2
    br = 2000
    assert n % br == 0

    def body(x_ref, d_ref, w_ref, o_ref):
        nrm = lax.rsqrt(jnp.maximum(d_ref[...], 1.0))
        xn = x_ref[...] * nrm
        o_ref[0, ...] = jnp.dot(
            xn, w_ref[:, :dhh], preferred_element_type=jnp.float32
        )
        o_ref[1, ...] = jnp.dot(
            xn, w_ref[:, dhh:], preferred_element_type=jnp.float32
        )

    return pl.pallas_call(
        body,
        grid=(n // br,),
        in_specs=[
            pl.BlockSpec((br, d_in), lambda i: (i, 0)),
            pl.BlockSpec((br, 1), lambda i: (i, 0)),
            pl.BlockSpec((d_in, d_h), lambda i: (0, 0)),
        ],
        out_specs=pl.BlockSpec((2, br, dhh), lambda i: (0, i, 0)),
        out_shape=jax.ShapeDtypeStruct((2, n, dhh), jnp.float32),
    )(x, deg_out, W)


def _tc_mid(agg_a, agg_b, deg_in, deg_out, b1, W2):
    n, dhh = agg_a.shape
    d_h = 2 * dhh
    d_out = W2.shape[1]
    br = 2560
    assert n % br == 0

    d_pad = 128

    def body(a_ref, bh_ref, di_ref, do_ref, b_ref, w_ref, o_ref):
        ni = lax.rsqrt(jnp.maximum(di_ref[...], 1.0))
        no = lax.rsqrt(jnp.maximum(do_ref[...], 1.0))
        h = jnp.concatenate([a_ref[...], bh_ref[...]], axis=1)
        h = jnp.maximum(h * ni + b_ref[...], 0.0) * no
        hw = jnp.dot(h, w_ref[...], preferred_element_type=jnp.float32)
        o_ref[...] = jnp.concatenate(
            [hw, jnp.zeros((br, d_pad - d_out), jnp.float32)], axis=1
        )

    return pl.pallas_call(
        body,
        grid=(n // br,),
        in_specs=[
            pl.BlockSpec((br, dhh), lambda i: (i, 0)),
            pl.BlockSpec((br, dhh), lambda i: (i, 0)),
            pl.BlockSpec((br, 1), lambda i: (i, 0)),
            pl.BlockSpec((br, 1), lambda i: (i, 0)),
            pl.BlockSpec((1, d_h), lambda i: (0, 0)),
            pl.BlockSpec((d_h, d_out), lambda i: (0, 0)),
        ],
        out_specs=pl.BlockSpec((br, d_pad), lambda i: (i, 0)),
        out_shape=jax.ShapeDtypeStruct((n, d_pad), jnp.float32),
    )(agg_a, agg_b, deg_in, deg_out, b1.reshape(1, d_h), W2)


def _tc_final(part_a, part_b, deg_in2, b2, n):
    npad, dpad = part_a.shape
    d_out = b2.shape[0]
    br = 2000
    assert n % br == 0

    def body(a_ref, bh_ref, di_ref, b_ref, o_ref):
        ni = lax.rsqrt(jnp.maximum(di_ref[...], 1.0))
        a2 = a_ref[...] + bh_ref[...]
        o_ref[...] = a2[:, :d_out] * ni + b_ref[...]

    return pl.pallas_call(
        body,
        grid=(n // br,),
        in_specs=[
            pl.BlockSpec((br, dpad), lambda i: (i, 0)),
            pl.BlockSpec((br, dpad), lambda i: (i, 0)),
            pl.BlockSpec((br, 1), lambda i: (i, 0)),
            pl.BlockSpec((1, d_out), lambda i: (0, 0)),
        ],
        out_specs=pl.BlockSpec((br, d_out), lambda i: (i, 0)),
        out_shape=jax.ShapeDtypeStruct((n, d_out), jnp.float32),
    )(part_a, part_b, deg_in2, b2.reshape(1, d_out))


def kernel(in_feat, edge_index, W1, b1, W2, b2):
    n, d_in = in_feat.shape
    e = edge_index.shape[1]
    d_h = W1.shape[1]
    d_out = W2.shape[1]
    assert e % (NC * NS) == 0 and d_h % 2 == 0 and d_out % 2 == 0

    n_pad = -(-n // (NS * C)) * (NS * C)

    src_arr = edge_index[0]
    dst_arr = edge_index[1]
    deg_out, deg_in = _degrees(src_arr, dst_arr, n_pad)
    deg_out = deg_out.reshape(n_pad, 1)
    deg_in = deg_in.reshape(n_pad, 1)

    hw1 = _tc_norm_matmul(in_feat, deg_out, W1)
    a1, a1b = _spmm_sc(hw1.reshape(2 * n, d_h // 2), src_arr, dst_arr, n_pad,
                       split_features=True, cs=64, nbuf=4)

    hw2 = _tc_mid(a1, a1b, deg_in, deg_out, b1, W2)
    p2a, p2b = _spmm_sc(hw2, src_arr, dst_arr, n_pad, split_features=False,
                        cs=64, nbuf=4)
    return _tc_final(p2a, p2b, deg_in, b2, n)
